# baseline (speedup 1.0000x reference)
import functools

import jax
import jax.numpy as jnp
from jax import lax
from jax.experimental import pallas as pl
from jax.experimental.pallas import tpu as pltpu
from jax.experimental.pallas import tpu_sc as plsc

_NUM_WORKERS = 32
_CHUNK = 128


@functools.partial(jax.jit, static_argnums=())
def _sc_gather_concat(fans_table, avgrating_table, fidx, aidx):
    b = fidx.shape[0] * fidx.shape[1] * fidx.shape[2]
    d = fans_table.shape[1]
    bpw = b // _NUM_WORKERS
    nch = bpw // _CHUNK
    mesh = plsc.VectorSubcoreMesh(core_axis_name="c", subcore_axis_name="s")

    @functools.partial(
        pl.kernel,
        mesh=mesh,
        compiler_params=pltpu.CompilerParams(use_tc_tiling_on_sc=False),
        out_type=jax.ShapeDtypeStruct((b, 2, d), jnp.float32),
        scratch_types=[
            pltpu.VMEM((nch, _CHUNK), jnp.int32),
            pltpu.VMEM((nch, _CHUNK), jnp.int32),
            pltpu.VMEM((bpw, d), jnp.float32),
            pltpu.VMEM((bpw, d), jnp.float32),
            pltpu.SemaphoreType.DMA,
        ],
    )
    def k(fans_hbm, avg_hbm, fidx_hbm, aidx_hbm, out_hbm,
          fidx_v, aidx_v, frows, arows, sem):
        wid = lax.axis_index("s") * 2 + lax.axis_index("c")
        base = wid * bpw
        pltpu.sync_copy(fidx_hbm.at[wid], fidx_v)
        pltpu.sync_copy(aidx_hbm.at[wid], aidx_v)
        copies = []
        for t in range(nch):
            sl = pl.ds(t * _CHUNK, _CHUNK)
            copies.append(
                pltpu.async_copy(fans_hbm.at[fidx_v.at[t]], frows.at[sl], sem))
            copies.append(
                pltpu.async_copy(avg_hbm.at[aidx_v.at[t]], arows.at[sl], sem))
        for c in copies:
            c.wait()
        pltpu.sync_copy(frows, out_hbm.at[pl.ds(base, bpw), 0])
        pltpu.sync_copy(arows, out_hbm.at[pl.ds(base, bpw), 1])

    return k(fans_table, avgrating_table, fidx, aidx)


def kernel(user_fea, fans_table, avgrating_table):
    b = user_fea.shape[0]
    d = fans_table.shape[1]
    bpw = b // _NUM_WORKERS
    nch = bpw // _CHUNK
    fidx = user_fea[:, 1].astype(jnp.int32).reshape(_NUM_WORKERS, nch, _CHUNK)
    aidx = user_fea[:, 2].astype(jnp.int32).reshape(_NUM_WORKERS, nch, _CHUNK)
    out = _sc_gather_concat(fans_table, avgrating_table, fidx, aidx)
    return out.reshape(b, 2 * d)


# speedup vs baseline: 1.3379x; 1.3379x over previous
import functools

import jax
import jax.numpy as jnp
from jax import lax
from jax.experimental import pallas as pl
from jax.experimental.pallas import tpu as pltpu
from jax.experimental.pallas import tpu_sc as plsc

_NUM_WORKERS = 32
_CHUNK = 128


@functools.partial(jax.jit, static_argnums=())
def _sc_gather_concat(fans_table, avgrating_table, fidx, aidx):
    b = fidx.shape[0] * fidx.shape[1] * fidx.shape[2]
    d = fans_table.shape[1]
    bpw = b // _NUM_WORKERS
    nch = bpw // _CHUNK
    mesh = plsc.VectorSubcoreMesh(core_axis_name="c", subcore_axis_name="s")

    @functools.partial(
        pl.kernel,
        mesh=mesh,
        compiler_params=pltpu.CompilerParams(use_tc_tiling_on_sc=False),
        out_type=jax.ShapeDtypeStruct((b, 2 * d), jnp.float32),
        scratch_types=[
            pltpu.VMEM((nch, _CHUNK), jnp.int32),
            pltpu.VMEM((nch, _CHUNK), jnp.int32),
            pltpu.VMEM((bpw, d), jnp.float32),
            pltpu.VMEM((bpw, d), jnp.float32),
            pltpu.SemaphoreType.DMA,
        ],
    )
    def k(fans_hbm, avg_hbm, fidx_hbm, aidx_hbm, out_hbm,
          fidx_v, aidx_v, frows, arows, sem):
        wid = lax.axis_index("s") * 2 + lax.axis_index("c")
        base = wid * bpw
        pltpu.sync_copy(fidx_hbm.at[wid], fidx_v)
        pltpu.sync_copy(aidx_hbm.at[wid], aidx_v)
        copies = []
        for t in range(nch):
            sl = pl.ds(t * _CHUNK, _CHUNK)
            copies.append(
                pltpu.async_copy(fans_hbm.at[fidx_v.at[t]], frows.at[sl], sem))
            copies.append(
                pltpu.async_copy(avg_hbm.at[aidx_v.at[t]], arows.at[sl], sem))
        for c in copies:
            c.wait()
        pltpu.sync_copy(frows, out_hbm.at[pl.ds(base, bpw), pl.ds(0, d)])
        pltpu.sync_copy(arows, out_hbm.at[pl.ds(base, bpw), pl.ds(d, d)])

    return k(fans_table, avgrating_table, fidx, aidx)


def kernel(user_fea, fans_table, avgrating_table):
    b = user_fea.shape[0]
    d = fans_table.shape[1]
    bpw = b // _NUM_WORKERS
    nch = bpw // _CHUNK
    fidx = user_fea[:, 1].astype(jnp.int32).reshape(_NUM_WORKERS, nch, _CHUNK)
    aidx = user_fea[:, 2].astype(jnp.int32).reshape(_NUM_WORKERS, nch, _CHUNK)
    return _sc_gather_concat(fans_table, avgrating_table, fidx, aidx)
